# Initial kernel scaffold; baseline (speedup 1.0000x reference)
import jax
import jax.numpy as jnp
from jax.experimental import pallas as pl


def kernel(weight, indices, padding_idx, scale_grad_by_freq, sparse, out):
    raise NotImplementedError("write your pallas kernel here")


# speedup vs baseline: 1.5613x; 1.5613x over previous
import functools

import jax
import jax.numpy as jnp
from jax import lax
from jax.experimental import pallas as pl
from jax.experimental.pallas import tpu as pltpu
from jax.experimental.pallas import tpu_sc as plsc


def _gather_kernel(N, D, NC, NW, n_per_w, CH, n_ch):
    mesh = plsc.VectorSubcoreMesh(core_axis_name="c", subcore_axis_name="s")

    @functools.partial(
        pl.kernel,
        mesh=mesh,
        out_type=jax.ShapeDtypeStruct((N, D), jnp.float32),
        scratch_types=[
            pltpu.VMEM((CH,), jnp.int32),
            pltpu.VMEM((CH, D), jnp.float32),
            pltpu.SemaphoreType.DMA,
        ],
        compiler_params=pltpu.CompilerParams(use_tc_tiling_on_sc=False),
    )
    def k(table_hbm, idx_hbm, out_hbm, idx_v, rows_v, sem):
        wid = lax.axis_index("s") * NC + lax.axis_index("c")
        base = wid * n_per_w

        def body(i, carry):
            off = base + i * CH
            pltpu.sync_copy(idx_hbm.at[pl.ds(off, CH)], idx_v)
            pltpu.async_copy(table_hbm.at[idx_v], rows_v, sem).wait()
            pltpu.sync_copy(rows_v, out_hbm.at[pl.ds(off, CH)])
            return carry

        lax.fori_loop(0, n_ch, body, 0)

    return k


def kernel(weight, indices, padding_idx, scale_grad_by_freq, sparse, out):
    B, F = indices.shape
    V, D = weight.shape
    N = B * F

    info = plsc.get_sparse_core_info()
    NC, NS = info.num_cores, info.num_subcores
    NW = NC * NS
    assert N % NW == 0
    n_per_w = N // NW

    CH = 1664
    assert n_per_w % CH == 0
    n_ch = n_per_w // CH

    idx_flat = indices.reshape(N)
    res = _gather_kernel(N, D, NC, NW, n_per_w, CH, n_ch)(weight, idx_flat)
    return res.reshape(B, F, D)


# speedup vs baseline: 1.5678x; 1.0041x over previous
import functools

import jax
import jax.numpy as jnp
from jax import lax
from jax.experimental import pallas as pl
from jax.experimental.pallas import tpu as pltpu
from jax.experimental.pallas import tpu_sc as plsc


def _gather_kernel(N, D, NC, NW, n_per_w, CH, n_ch):
    mesh = plsc.VectorSubcoreMesh(core_axis_name="c", subcore_axis_name="s")
    NBUF = 2

    @functools.partial(
        pl.kernel,
        mesh=mesh,
        out_type=jax.ShapeDtypeStruct((N, D), jnp.float32),
        scratch_types=[
            [pltpu.VMEM((CH,), jnp.int32) for _ in range(NBUF)],
            [pltpu.VMEM((CH, D), jnp.float32) for _ in range(NBUF)],
            [pltpu.SemaphoreType.DMA for _ in range(NBUF)],
            [pltpu.SemaphoreType.DMA for _ in range(NBUF)],
            [pltpu.SemaphoreType.DMA for _ in range(NBUF)],
        ],
        compiler_params=pltpu.CompilerParams(use_tc_tiling_on_sc=False),
    )
    def k(table_hbm, idx_hbm, out_hbm, idx_v, rows_v, sem_i, sem_g, sem_o):
        wid = lax.axis_index("s") * NC + lax.axis_index("c")
        base = wid * n_per_w

        idx_cp = [None] * n_ch
        out_cp = [None] * n_ch

        for b in range(min(NBUF, n_ch)):
            idx_cp[b] = pltpu.async_copy(
                idx_hbm.at[pl.ds(base + b * CH, CH)], idx_v[b], sem_i[b])

        for g in range(n_ch):
            b = g % NBUF
            off = base + g * CH
            idx_cp[g].wait()
            if g >= NBUF:
                out_cp[g - NBUF].wait()
            gather = pltpu.async_copy(table_hbm.at[idx_v[b]], rows_v[b], sem_g[b])
            gather.wait()
            out_cp[g] = pltpu.async_copy(
                rows_v[b], out_hbm.at[pl.ds(off, CH)], sem_o[b])
            nxt = g + NBUF
            if nxt < n_ch:
                idx_cp[nxt] = pltpu.async_copy(
                    idx_hbm.at[pl.ds(base + nxt * CH, CH)], idx_v[b], sem_i[b])

        for g in range(max(0, n_ch - NBUF), n_ch):
            out_cp[g].wait()

    return k


def kernel(weight, indices, padding_idx, scale_grad_by_freq, sparse, out):
    B, F = indices.shape
    V, D = weight.shape
    N = B * F

    info = plsc.get_sparse_core_info()
    NC, NS = info.num_cores, info.num_subcores
    NW = NC * NS
    assert N % NW == 0
    n_per_w = N // NW

    CH = 1664
    assert n_per_w % CH == 0
    n_ch = n_per_w // CH

    idx_flat = indices.reshape(N)
    res = _gather_kernel(N, D, NC, NW, n_per_w, CH, n_ch)(weight, idx_flat)
    return res.reshape(B, F, D)


# speedup vs baseline: 1.7516x; 1.1173x over previous
import functools

import jax
import jax.numpy as jnp
from jax import lax
from jax.experimental import pallas as pl
from jax.experimental.pallas import tpu as pltpu
from jax.experimental.pallas import tpu_sc as plsc


def _tc_linearize(wt, V, D):
    R = 1024
    rows = V * D // 128
    nb = -(-rows // R)

    def body(in_ref, out_ref):
        x = in_ref[...]
        y = jnp.transpose(x, (1, 0))
        z = y.reshape(R, 128 // D, D)
        out_ref[...] = jnp.concatenate(
            [z[:, q, :] for q in range(128 // D)], axis=1)

    return pl.pallas_call(
        body,
        grid=(nb,),
        in_specs=[pl.BlockSpec((D, (128 // D) * R), lambda i: (0, i))],
        out_specs=pl.BlockSpec((R, 128), lambda i: (i, 0)),
        out_shape=jax.ShapeDtypeStruct((rows, 128), jnp.float32),
    )(wt)


def _gather_kernel(N, D, NC, NW, n_per_w, CH, n_ch):
    mesh = plsc.VectorSubcoreMesh(core_axis_name="c", subcore_axis_name="s")
    NBUF = 2

    @functools.partial(
        pl.kernel,
        mesh=mesh,
        out_type=jax.ShapeDtypeStruct((N, D), jnp.float32),
        scratch_types=[
            [pltpu.VMEM((CH,), jnp.int32) for _ in range(NBUF)],
            [pltpu.VMEM((CH, D), jnp.float32) for _ in range(NBUF)],
            [pltpu.SemaphoreType.DMA for _ in range(NBUF)],
            [pltpu.SemaphoreType.DMA for _ in range(NBUF)],
            [pltpu.SemaphoreType.DMA for _ in range(NBUF)],
        ],
        compiler_params=pltpu.CompilerParams(use_tc_tiling_on_sc=False),
    )
    def k(table_hbm, idx_hbm, out_hbm, idx_v, rows_v, sem_i, sem_g, sem_o):
        wid = lax.axis_index("s") * NC + lax.axis_index("c")
        base = wid * n_per_w

        idx_cp = [None] * n_ch
        out_cp = [None] * n_ch

        for b in range(min(NBUF, n_ch)):
            idx_cp[b] = pltpu.async_copy(
                idx_hbm.at[pl.ds(base + b * CH, CH)], idx_v[b], sem_i[b])

        for g in range(n_ch):
            b = g % NBUF
            off = base + g * CH
            idx_cp[g].wait()
            if g >= NBUF:
                out_cp[g - NBUF].wait()
            gather = pltpu.async_copy(table_hbm.at[idx_v[b]], rows_v[b], sem_g[b])
            gather.wait()
            out_cp[g] = pltpu.async_copy(
                rows_v[b], out_hbm.at[pl.ds(off, CH)], sem_o[b])
            nxt = g + NBUF
            if nxt < n_ch:
                idx_cp[nxt] = pltpu.async_copy(
                    idx_hbm.at[pl.ds(base + nxt * CH, CH)], idx_v[b], sem_i[b])

        for g in range(max(0, n_ch - NBUF), n_ch):
            out_cp[g].wait()

    return k


def kernel(weight, indices, padding_idx, scale_grad_by_freq, sparse, out):
    B, F = indices.shape
    V, D = weight.shape
    N = B * F

    info = plsc.get_sparse_core_info()
    NC, NS = info.num_cores, info.num_subcores
    NW = NC * NS
    assert N % NW == 0
    n_per_w = N // NW

    CH = 1664
    assert n_per_w % CH == 0
    n_ch = n_per_w // CH

    idx_flat = indices.reshape(N)
    w128 = _tc_linearize(jnp.swapaxes(weight, 0, 1), V, D)
    res = _gather_kernel(N, D, NC, NW, n_per_w, CH, n_ch)(
        w128.reshape(V, D), idx_flat)
    return res.reshape(B, F, D)
